# baseline (speedup 1.0000x reference)
import functools

import jax
import jax.numpy as jnp
from jax import lax
from jax.experimental import pallas as pl
from jax.experimental.pallas import tpu as pltpu
from jax.experimental.pallas import tpu_sc as plsc

F = 16
B = 16384
NC = 2
NS = 16
NW = NC * NS
BPW = B // NW


def _sc_body(users_h, items_h, ug_h, ig_h, um_h, im_h,
             gmf_o, um_o, im_o,
             uidx, iidx, ug_v, ig_v, um_v, im_v, sem):
    wid = lax.axis_index("s") * NC + lax.axis_index("c")
    base = wid * BPW
    pltpu.sync_copy(users_h.at[pl.ds(base, BPW)], uidx)
    pltpu.sync_copy(items_h.at[pl.ds(base, BPW)], iidx)
    c0 = pltpu.async_copy(ug_h.at[uidx], ug_v, sem)
    c1 = pltpu.async_copy(ig_h.at[iidx], ig_v, sem)
    c2 = pltpu.async_copy(um_h.at[uidx], um_v, sem)
    c3 = pltpu.async_copy(im_h.at[iidx], im_v, sem)
    c0.wait()
    c1.wait()

    def mul4(i, c):
        r = i * 4
        for k in range(4):
            ug_v[r + k] = ug_v[r + k] * ig_v[r + k]
        return c

    lax.fori_loop(0, BPW // 4, mul4, 0)
    pltpu.sync_copy(ug_v, gmf_o.at[pl.ds(base, BPW)])
    c2.wait()
    pltpu.sync_copy(um_v, um_o.at[pl.ds(base, BPW)])
    c3.wait()
    pltpu.sync_copy(im_v, im_o.at[pl.ds(base, BPW)])


_sc_gather = functools.partial(
    pl.kernel,
    mesh=plsc.VectorSubcoreMesh(core_axis_name="c", subcore_axis_name="s"),
    compiler_params=pltpu.CompilerParams(use_tc_tiling_on_sc=False),
    out_type=[
        jax.ShapeDtypeStruct((B, F), jnp.float32),
        jax.ShapeDtypeStruct((B, F), jnp.float32),
        jax.ShapeDtypeStruct((B, F), jnp.float32),
    ],
    scratch_types=[
        pltpu.VMEM((BPW,), jnp.int32),
        pltpu.VMEM((BPW,), jnp.int32),
        pltpu.VMEM((BPW, F), jnp.float32),
        pltpu.VMEM((BPW, F), jnp.float32),
        pltpu.VMEM((BPW, F), jnp.float32),
        pltpu.VMEM((BPW, F), jnp.float32),
        pltpu.SemaphoreType.DMA,
    ],
)(_sc_body)


BM = 2048


def _tc_body(gmf_ref, um_ref, im_ref, w1_ref, b1_ref, w2_ref, b2_ref,
             wog_ref, woh_ref, bo_ref, out_ref):
    mlp_in = jnp.concatenate([um_ref[...], im_ref[...]], axis=1)
    h = jnp.dot(mlp_in, w1_ref[...], preferred_element_type=jnp.float32)
    h = jnp.maximum(h + b1_ref[...], 0.0)
    h = jnp.dot(h, w2_ref[...], preferred_element_type=jnp.float32)
    h = jnp.maximum(h + b2_ref[...], 0.0)
    s = jnp.dot(gmf_ref[...], wog_ref[...], preferred_element_type=jnp.float32)
    s = s + jnp.dot(h, woh_ref[...], preferred_element_type=jnp.float32)
    out_ref[...] = s + bo_ref[...]


def _tc_mlp(gmf, um, im, W1, b1, W2, b2, Wo, bo):
    grid = (B // BM,)
    full = lambda shape: pl.BlockSpec(shape, lambda i: (0, 0))
    return pl.pallas_call(
        _tc_body,
        grid=grid,
        in_specs=[
            pl.BlockSpec((BM, F), lambda i: (i, 0)),
            pl.BlockSpec((BM, F), lambda i: (i, 0)),
            pl.BlockSpec((BM, F), lambda i: (i, 0)),
            full((2 * F, 2 * F)),
            full((1, 2 * F)),
            full((2 * F, F)),
            full((1, F)),
            full((F, 1)),
            full((F, 1)),
            full((1, 1)),
        ],
        out_specs=pl.BlockSpec((BM, 1), lambda i: (i, 0)),
        out_shape=jax.ShapeDtypeStruct((B, 1), jnp.float32),
    )(gmf, um, im, W1, b1.reshape(1, -1), W2, b2.reshape(1, -1),
      Wo[:F], Wo[F:], bo.reshape(1, 1))


def kernel(users, items, user_gmf, item_gmf, user_mlp, item_mlp,
           W1, b1, W2, b2, Wo, bo):
    users = users.astype(jnp.int32)
    items = items.astype(jnp.int32)
    gmf, um, im = _sc_gather(users, items, user_gmf, item_gmf,
                             user_mlp, item_mlp)
    scores = _tc_mlp(gmf, um, im, W1, b1, W2, b2, Wo, bo)
    return scores[:, 0]
